# Initial kernel scaffold; baseline (speedup 1.0000x reference)
import jax
import jax.numpy as jnp
from jax.experimental import pallas as pl


def kernel(x, edge_index, is_cluster, W1, b1, W2, b2):
    raise NotImplementedError("write your pallas kernel here")


# speedup vs baseline: 8.5406x; 8.5406x over previous
import functools

import jax
import jax.numpy as jnp
from jax import lax
from jax.experimental import pallas as pl
from jax.experimental.pallas import tpu as pltpu
from jax.experimental.pallas import tpu_sc as plsc

_N = 10000
_D = 128
_E = 320000
_K = 3

_NCORES = 2
_NSUB = 16
_NTILES = _NCORES * _NSUB

_CHUNK = 128
_CPT = 79
_EPT = _CHUNK * _CPT
_EPAD = _EPT * _NTILES
_NP = 10112
_RPT = _NP // _NSUB

_ROWS_BLK = 2000
_GRID = _N // _ROWS_BLK


def _make_sc_scatter():
    mesh = plsc.VectorSubcoreMesh(core_axis_name="c", subcore_axis_name="s")

    @functools.partial(
        pl.kernel,
        out_type=jax.ShapeDtypeStruct((_NCORES, _NP, _D), jnp.float32),
        mesh=mesh,
        scratch_types=[
            pltpu.VMEM((_CHUNK,), jnp.int32),
            pltpu.VMEM((_CHUNK,), jnp.int32),
            pltpu.VMEM((_CHUNK, _D), jnp.float32),
            pltpu.VMEM_SHARED((_NP, _D), jnp.float32),
            pltpu.SemaphoreType.DMA,
        ],
    )
    def sc_scatter(g_hbm, row_hbm, col_hbm, zeros_hbm, out_hbm,
                   ridx, cidx, rows, acc, sem):
        cid = lax.axis_index("c")
        sid = lax.axis_index("s")
        tid = cid * _NSUB + sid
        r0 = pl.multiple_of(sid * _RPT, 8)
        pltpu.sync_copy(zeros_hbm.at[pl.ds(r0, _RPT)], acc.at[pl.ds(r0, _RPT)])
        plsc.subcore_barrier()

        def body(j, carry):
            base = pl.multiple_of(tid * _EPT + j * _CHUNK, _CHUNK)
            pltpu.sync_copy(row_hbm.at[pl.ds(base, _CHUNK)], ridx)
            pltpu.sync_copy(col_hbm.at[pl.ds(base, _CHUNK)], cidx)
            pltpu.async_copy(g_hbm.at[ridx], rows, sem).wait()
            pltpu.sync_copy(rows, acc.at[cidx], add=True)
            return carry

        lax.fori_loop(0, _CPT, body, 0)
        plsc.subcore_barrier()
        pltpu.sync_copy(acc.at[pl.ds(r0, _RPT)],
                        out_hbm.at[cid, pl.ds(r0, _RPT)])

    return sc_scatter


def _make_sc_degree():
    mesh = plsc.VectorSubcoreMesh(core_axis_name="c", subcore_axis_name="s")

    @functools.partial(
        pl.kernel,
        out_type=jax.ShapeDtypeStruct((_NCORES, _NP, _D), jnp.float32),
        mesh=mesh,
        scratch_types=[
            pltpu.VMEM((_CHUNK,), jnp.int32),
            pltpu.VMEM((_CHUNK, _D), jnp.float32),
            pltpu.VMEM_SHARED((_NP, _D), jnp.float32),
        ],
    )
    def sc_degree(col_hbm, ones_hbm, zeros_hbm, out_hbm, cidx, ones_v, acc):
        cid = lax.axis_index("c")
        sid = lax.axis_index("s")
        tid = cid * _NSUB + sid
        r0 = pl.multiple_of(sid * _RPT, 8)
        pltpu.sync_copy(zeros_hbm.at[pl.ds(r0, _RPT)], acc.at[pl.ds(r0, _RPT)])
        pltpu.sync_copy(ones_hbm, ones_v)
        plsc.subcore_barrier()

        def body(j, carry):
            base = pl.multiple_of(tid * _EPT + j * _CHUNK, _CHUNK)
            pltpu.sync_copy(col_hbm.at[pl.ds(base, _CHUNK)], cidx)
            pltpu.sync_copy(ones_v, acc.at[cidx], add=True)
            return carry

        lax.fori_loop(0, _CPT, body, 0)
        plsc.subcore_barrier()
        pltpu.sync_copy(acc.at[pl.ds(r0, _RPT)],
                        out_hbm.at[cid, pl.ds(r0, _RPT)])

    return sc_degree


_sc_scatter = _make_sc_scatter()
_sc_degree = _make_sc_degree()


def _prep_body(degp_ref, x_ref, dinv_ref, dinv2_ref, g0_ref):
    deg = degp_ref[0, :, 0:1] + degp_ref[1, :, 0:1] + 1.0
    dv = lax.rsqrt(deg)
    dinv_ref[...] = dv
    dinv2_ref[...] = dv * dv
    g0_ref[...] = dv * x_ref[...]


def _prep(degp, x):
    return pl.pallas_call(
        _prep_body,
        grid=(_GRID,),
        in_specs=[
            pl.BlockSpec((_NCORES, _ROWS_BLK, _D), lambda i: (0, i, 0)),
            pl.BlockSpec((_ROWS_BLK, _D), lambda i: (i, 0)),
        ],
        out_specs=[
            pl.BlockSpec((_ROWS_BLK, 1), lambda i: (i, 0)),
            pl.BlockSpec((_ROWS_BLK, 1), lambda i: (i, 0)),
            pl.BlockSpec((_ROWS_BLK, _D), lambda i: (i, 0)),
        ],
        out_shape=[
            jax.ShapeDtypeStruct((_N, 1), jnp.float32),
            jax.ShapeDtypeStruct((_N, 1), jnp.float32),
            jax.ShapeDtypeStruct((_N, _D), jnp.float32),
        ],
    )(degp, x)


def _merge_body(p_ref, g_ref, dinv2_ref, out_ref):
    out_ref[...] = dinv2_ref[...] * (p_ref[0] + p_ref[1] + g_ref[...])


def _merge(p, g, dinv2):
    return pl.pallas_call(
        _merge_body,
        grid=(_GRID,),
        in_specs=[
            pl.BlockSpec((_NCORES, _ROWS_BLK, _D), lambda i: (0, i, 0)),
            pl.BlockSpec((_ROWS_BLK, _D), lambda i: (i, 0)),
            pl.BlockSpec((_ROWS_BLK, 1), lambda i: (i, 0)),
        ],
        out_specs=pl.BlockSpec((_ROWS_BLK, _D), lambda i: (i, 0)),
        out_shape=jax.ShapeDtypeStruct((_N, _D), jnp.float32),
    )(p, g, dinv2)


def _final_body(p_ref, g_ref, dinv_ref, w1_ref, b1_ref, w2_ref, b2_ref, out_ref):
    h = dinv_ref[...] * (p_ref[0] + p_ref[1] + g_ref[...])
    z = lax.dot_general(h, w1_ref[...], (((1,), (1,)), ((), ())),
                        preferred_element_type=jnp.float32) + b1_ref[...]
    z = jnp.maximum(z, 0.0)
    z = lax.dot_general(z, w2_ref[...], (((1,), (1,)), ((), ())),
                        preferred_element_type=jnp.float32) + b2_ref[...]
    m = jnp.max(z, axis=1, keepdims=True)
    e = z - m
    out_ref[...] = e - jnp.log(jnp.sum(jnp.exp(e), axis=1, keepdims=True))


def _final(p, g, dinv, w1, b1, w2, b2):
    return pl.pallas_call(
        _final_body,
        grid=(_GRID,),
        in_specs=[
            pl.BlockSpec((_NCORES, _ROWS_BLK, _D), lambda i: (0, i, 0)),
            pl.BlockSpec((_ROWS_BLK, _D), lambda i: (i, 0)),
            pl.BlockSpec((_ROWS_BLK, 1), lambda i: (i, 0)),
            pl.BlockSpec((_D, _D), lambda i: (0, 0)),
            pl.BlockSpec((1, _D), lambda i: (0, 0)),
            pl.BlockSpec((_D, _D), lambda i: (0, 0)),
            pl.BlockSpec((1, _D), lambda i: (0, 0)),
        ],
        out_specs=pl.BlockSpec((_ROWS_BLK, _D), lambda i: (i, 0)),
        out_shape=jax.ShapeDtypeStruct((_N, _D), jnp.float32),
    )(p, g, dinv, w1, b1, w2, b2)


def kernel(x, edge_index, is_cluster, W1, b1, W2, b2):
    row = edge_index[0]
    col = edge_index[1]
    pad = _EPAD - _E
    rowp = jnp.concatenate([row, jnp.zeros((pad,), jnp.int32)])
    colp = jnp.concatenate([col, jnp.full((pad,), _N, jnp.int32)])
    zeros_d = jnp.zeros((_NP, _D), jnp.float32)
    ones_d = jnp.ones((_CHUNK, _D), jnp.float32)

    degp = _sc_degree(colp, ones_d, zeros_d)
    dinv, dinv2, g = _prep(degp, x)
    p = None
    for k in range(_K):
        p = _sc_scatter(g, rowp, colp, zeros_d)
        if k < _K - 1:
            g = _merge(p, g, dinv2)
    return _final(p, g, dinv, W1,
                  b1.reshape(1, _D), W2, b2.reshape(1, _D))
